# Initial kernel scaffold; baseline (speedup 1.0000x reference)
import functools

import jax
import jax.numpy as jnp
from jax import lax
from jax.experimental import pallas as pl
from jax.experimental.pallas import tpu as pltpu
from jax.experimental.pallas import tpu_sc as plsc

_NC = 2
_NS = 16
_NW = _NC * _NS
_LANES = 16
_CHUNK = 80
_ROWBLK = 1000


def _sc_mesh():
    return plsc.VectorSubcoreMesh(core_axis_name="c", subcore_axis_name="s")


def _sc_degree(dst2d, n):
    nrows = dst2d.shape[0]
    rpt = nrows // _NW
    nzb = n // _CHUNK
    kblocks = (nzb + _NS - 1) // _NS

    @functools.partial(
        pl.kernel,
        out_type=jax.ShapeDtypeStruct((_NC, n, _LANES), jnp.float32),
        mesh=_sc_mesh(),
        scratch_types=[
            pltpu.VMEM((rpt, _CHUNK), jnp.int32),
            pltpu.VMEM((_CHUNK, _LANES), jnp.float32),
            pltpu.VMEM((_CHUNK, _LANES), jnp.float32),
            pltpu.VMEM_SHARED((n, _LANES), jnp.float32),
        ],
    )
    def deg_kernel(dst_hbm, out_hbm, idx_v, ones_v, zero_v, acc_sh):
        cid = lax.axis_index("c")
        sid = lax.axis_index("s")
        wid = cid * _NS + sid

        @pl.loop(0, _CHUNK)
        def _(i):
            ones_v[i, :] = jnp.full((_LANES,), 1.0, jnp.float32)
            zero_v[i, :] = jnp.zeros((_LANES,), jnp.float32)

        for k in range(kblocks):
            b = sid + _NS * k

            @pl.when(b < nzb)
            def _():
                pltpu.sync_copy(zero_v, acc_sh.at[pl.ds(b * _CHUNK, _CHUNK)])

        plsc.subcore_barrier()

        pltpu.sync_copy(dst_hbm.at[pl.ds(wid * rpt, rpt)], idx_v)

        @pl.loop(0, rpt)
        def _(j):
            pltpu.sync_copy(ones_v, acc_sh.at[idx_v.at[j]], add=True)

        plsc.subcore_barrier()

        for k in range(kblocks):
            b = sid + _NS * k

            @pl.when(b < nzb)
            def _():
                pltpu.sync_copy(
                    acc_sh.at[pl.ds(b * _CHUNK, _CHUNK)],
                    out_hbm.at[cid].at[pl.ds(b * _CHUNK, _CHUNK)],
                )

    return deg_kernel(dst2d)


def _sc_propagate(y, src2d, dst2d):
    n, d = y.shape
    nrows = src2d.shape[0]
    rpt = nrows // _NW
    nzb = n // _CHUNK
    kblocks = (nzb + _NS - 1) // _NS

    @functools.partial(
        pl.kernel,
        out_type=jax.ShapeDtypeStruct((_NC, n, d), jnp.float32),
        mesh=_sc_mesh(),
        scratch_types=[
            pltpu.VMEM((rpt, _CHUNK), jnp.int32),
            pltpu.VMEM((rpt, _CHUNK), jnp.int32),
            pltpu.VMEM((_CHUNK, d), jnp.float32),
            pltpu.VMEM_SHARED((n, d), jnp.float32),
        ],
    )
    def prop_kernel(y_hbm, src_hbm, dst_hbm, out_hbm, src_v, dst_v, rows_v, acc_sh):
        cid = lax.axis_index("c")
        sid = lax.axis_index("s")
        wid = cid * _NS + sid

        @pl.loop(0, _CHUNK)
        def _(i):
            for j in range(d // _LANES):
                rows_v[i, pl.ds(j * _LANES, _LANES)] = jnp.zeros(
                    (_LANES,), jnp.float32
                )

        for k in range(kblocks):
            b = sid + _NS * k

            @pl.when(b < nzb)
            def _():
                pltpu.sync_copy(rows_v, acc_sh.at[pl.ds(b * _CHUNK, _CHUNK)])

        plsc.subcore_barrier()

        pltpu.sync_copy(src_hbm.at[pl.ds(wid * rpt, rpt)], src_v)
        pltpu.sync_copy(dst_hbm.at[pl.ds(wid * rpt, rpt)], dst_v)

        @pl.loop(0, rpt)
        def _(j):
            pltpu.sync_copy(y_hbm.at[src_v.at[j]], rows_v)
            pltpu.sync_copy(rows_v, acc_sh.at[dst_v.at[j]], add=True)

        plsc.subcore_barrier()

        for k in range(kblocks):
            b = sid + _NS * k

            @pl.when(b < nzb)
            def _():
                pltpu.sync_copy(
                    acc_sh.at[pl.ds(b * _CHUNK, _CHUNK)],
                    out_hbm.at[cid].at[pl.ds(b * _CHUNK, _CHUNK)],
                )

    return prop_kernel(y, src2d, dst2d)


def _dis_block(degp):
    deg = degp[0, :, 0:1] + degp[1, :, 0:1] + 1.0
    return lax.rsqrt(deg)


def _tc_y1(x, w1, degp):
    n, d = x.shape

    def body(x_ref, w_ref, dp_ref, o_ref):
        dis = _dis_block(dp_ref[...])
        o_ref[...] = (
            jnp.dot(x_ref[...], w_ref[...], preferred_element_type=jnp.float32)
            * dis
        )

    return pl.pallas_call(
        body,
        grid=(n // _ROWBLK,),
        in_specs=[
            pl.BlockSpec((_ROWBLK, d), lambda i: (i, 0)),
            pl.BlockSpec((d, w1.shape[1]), lambda i: (0, 0)),
            pl.BlockSpec((2, _ROWBLK, _LANES), lambda i: (0, i, 0)),
        ],
        out_specs=pl.BlockSpec((_ROWBLK, w1.shape[1]), lambda i: (i, 0)),
        out_shape=jax.ShapeDtypeStruct((n, w1.shape[1]), jnp.float32),
    )(x, w1, degp)


def _tc_mid(parts, y, degp, b_row, w_next):
    n, d = y.shape
    dn = w_next.shape[1]

    def body(p_ref, y_ref, dp_ref, b_ref, w_ref, o_ref):
        dis = _dis_block(dp_ref[...])
        p = p_ref[...]
        raw = p[0] + p[1] + y_ref[...]
        h = jnp.maximum(dis * raw + b_ref[...], 0.0)
        o_ref[...] = (
            jnp.dot(h, w_ref[...], preferred_element_type=jnp.float32) * dis
        )

    return pl.pallas_call(
        body,
        grid=(n // _ROWBLK,),
        in_specs=[
            pl.BlockSpec((2, _ROWBLK, d), lambda i: (0, i, 0)),
            pl.BlockSpec((_ROWBLK, d), lambda i: (i, 0)),
            pl.BlockSpec((2, _ROWBLK, _LANES), lambda i: (0, i, 0)),
            pl.BlockSpec((1, d), lambda i: (0, 0)),
            pl.BlockSpec((d, dn), lambda i: (0, 0)),
        ],
        out_specs=pl.BlockSpec((_ROWBLK, dn), lambda i: (i, 0)),
        out_shape=jax.ShapeDtypeStruct((n, dn), jnp.float32),
    )(parts, y, degp, b_row, w_next)


def _tc_fin(parts, y, degp, b_row, wc, bc_row):
    n, d = y.shape
    dn = wc.shape[1]

    def body(p_ref, y_ref, dp_ref, b_ref, w_ref, bc_ref, o_ref):
        dis = _dis_block(dp_ref[...])
        p = p_ref[...]
        raw = p[0] + p[1] + y_ref[...]
        h = jnp.maximum(dis * raw + b_ref[...], 0.0)
        o_ref[...] = (
            jnp.dot(h, w_ref[...], preferred_element_type=jnp.float32)
            + bc_ref[...]
        )

    return pl.pallas_call(
        body,
        grid=(n // _ROWBLK,),
        in_specs=[
            pl.BlockSpec((2, _ROWBLK, d), lambda i: (0, i, 0)),
            pl.BlockSpec((_ROWBLK, d), lambda i: (i, 0)),
            pl.BlockSpec((2, _ROWBLK, _LANES), lambda i: (0, i, 0)),
            pl.BlockSpec((1, d), lambda i: (0, 0)),
            pl.BlockSpec((d, dn), lambda i: (0, 0)),
            pl.BlockSpec((1, dn), lambda i: (0, 0)),
        ],
        out_specs=pl.BlockSpec((_ROWBLK, dn), lambda i: (i, 0)),
        out_shape=jax.ShapeDtypeStruct((n, dn), jnp.float32),
    )(parts, y, degp, b_row, wc, bc_row)


def kernel(x, edge_index, W1, b1, W2, b2, Wc, bc):
    n, _ = x.shape
    src2d = edge_index[0].astype(jnp.int32).reshape(-1, _CHUNK)
    dst2d = edge_index[1].astype(jnp.int32).reshape(-1, _CHUNK)

    degp = _sc_degree(dst2d, n)
    y1 = _tc_y1(x, W1, degp)
    p1 = _sc_propagate(y1, src2d, dst2d)
    y2 = _tc_mid(p1, y1, degp, b1.reshape(1, -1), W2)
    p2 = _sc_propagate(y2, src2d, dst2d)
    return _tc_fin(p2, y2, degp, b2.reshape(1, -1), Wc, bc.reshape(1, -1))


# speedup vs baseline: 19.5343x; 19.5343x over previous
import functools

import jax
import jax.numpy as jnp
from jax import lax
from jax.experimental import pallas as pl
from jax.experimental.pallas import tpu as pltpu
from jax.experimental.pallas import tpu_sc as plsc

_NC = 2
_NS = 16
_NW = _NC * _NS
_LANES = 16
_CHUNK = 80
_ROWBLK = 1000


def _sc_mesh():
    return plsc.VectorSubcoreMesh(core_axis_name="c", subcore_axis_name="s")


def _sc_degree(dst3d, n):
    rpt = dst3d.shape[1]
    nzb = n // _CHUNK
    kblocks = (nzb + _NS - 1) // _NS

    @functools.partial(
        pl.kernel,
        out_type=jax.ShapeDtypeStruct((_NC, n, _LANES), jnp.float32),
        mesh=_sc_mesh(),
        scratch_types=[
            pltpu.VMEM((rpt, _CHUNK), jnp.int32),
            pltpu.VMEM((_CHUNK, _LANES), jnp.float32),
            pltpu.VMEM((_CHUNK, _LANES), jnp.float32),
            pltpu.VMEM_SHARED((n, _LANES), jnp.float32),
        ],
    )
    def deg_kernel(dst_hbm, out_hbm, idx_v, ones_v, zero_v, acc_sh):
        cid = lax.axis_index("c")
        sid = lax.axis_index("s")
        wid = cid * _NS + sid

        @pl.loop(0, _CHUNK)
        def _(i):
            ones_v[i, :] = jnp.full((_LANES,), 1.0, jnp.float32)
            zero_v[i, :] = jnp.zeros((_LANES,), jnp.float32)

        for k in range(kblocks):
            b = sid + _NS * k

            @pl.when(b < nzb)
            def _():
                pltpu.sync_copy(zero_v, acc_sh.at[pl.ds(b * _CHUNK, _CHUNK)])

        plsc.subcore_barrier()

        pltpu.sync_copy(dst_hbm.at[wid], idx_v)

        @pl.loop(0, rpt)
        def _(j):
            pltpu.sync_copy(ones_v, acc_sh.at[idx_v.at[j]], add=True)

        plsc.subcore_barrier()

        for k in range(kblocks):
            b = sid + _NS * k

            @pl.when(b < nzb)
            def _():
                pltpu.sync_copy(
                    acc_sh.at[pl.ds(b * _CHUNK, _CHUNK)],
                    out_hbm.at[cid].at[pl.ds(b * _CHUNK, _CHUNK)],
                )

    return deg_kernel(dst3d)


def _sc_propagate(y, src3d, dst3d):
    n, d = y.shape
    rpt = src3d.shape[1]
    nzb = n // _CHUNK
    kblocks = (nzb + _NS - 1) // _NS

    @functools.partial(
        pl.kernel,
        out_type=jax.ShapeDtypeStruct((_NC, n, d), jnp.float32),
        mesh=_sc_mesh(),
        scratch_types=[
            pltpu.VMEM((rpt, _CHUNK), jnp.int32),
            pltpu.VMEM((rpt, _CHUNK), jnp.int32),
            pltpu.VMEM((_CHUNK, d), jnp.float32),
            pltpu.VMEM_SHARED((n, d), jnp.float32),
        ],
    )
    def prop_kernel(y_hbm, src_hbm, dst_hbm, out_hbm, src_v, dst_v, rows_v, acc_sh):
        cid = lax.axis_index("c")
        sid = lax.axis_index("s")
        wid = cid * _NS + sid

        @pl.loop(0, _CHUNK)
        def _(i):
            for j in range(d // _LANES):
                rows_v[i, pl.ds(j * _LANES, _LANES)] = jnp.zeros(
                    (_LANES,), jnp.float32
                )

        for k in range(kblocks):
            b = sid + _NS * k

            @pl.when(b < nzb)
            def _():
                pltpu.sync_copy(rows_v, acc_sh.at[pl.ds(b * _CHUNK, _CHUNK)])

        plsc.subcore_barrier()

        pltpu.sync_copy(src_hbm.at[wid], src_v)
        pltpu.sync_copy(dst_hbm.at[wid], dst_v)

        @pl.loop(0, rpt)
        def _(j):
            pltpu.sync_copy(y_hbm.at[src_v.at[j]], rows_v)
            pltpu.sync_copy(rows_v, acc_sh.at[dst_v.at[j]], add=True)

        plsc.subcore_barrier()

        for k in range(kblocks):
            b = sid + _NS * k

            @pl.when(b < nzb)
            def _():
                pltpu.sync_copy(
                    acc_sh.at[pl.ds(b * _CHUNK, _CHUNK)],
                    out_hbm.at[cid].at[pl.ds(b * _CHUNK, _CHUNK)],
                )

    return prop_kernel(y, src3d, dst3d)


def _dis_block(degp):
    deg = degp[0, :, 0:1] + degp[1, :, 0:1] + 1.0
    return lax.rsqrt(deg)


def _tc_y1(x, w1, degp):
    n, d = x.shape

    def body(x_ref, w_ref, dp_ref, o_ref):
        dis = _dis_block(dp_ref[...])
        o_ref[...] = (
            jnp.dot(x_ref[...], w_ref[...], preferred_element_type=jnp.float32)
            * dis
        )

    return pl.pallas_call(
        body,
        grid=(n // _ROWBLK,),
        in_specs=[
            pl.BlockSpec((_ROWBLK, d), lambda i: (i, 0)),
            pl.BlockSpec((d, w1.shape[1]), lambda i: (0, 0)),
            pl.BlockSpec((2, _ROWBLK, _LANES), lambda i: (0, i, 0)),
        ],
        out_specs=pl.BlockSpec((_ROWBLK, w1.shape[1]), lambda i: (i, 0)),
        out_shape=jax.ShapeDtypeStruct((n, w1.shape[1]), jnp.float32),
    )(x, w1, degp)


def _tc_mid(parts, y, degp, b_row, w_next):
    n, d = y.shape
    dn = w_next.shape[1]

    def body(p_ref, y_ref, dp_ref, b_ref, w_ref, o_ref):
        dis = _dis_block(dp_ref[...])
        p = p_ref[...]
        raw = p[0] + p[1] + y_ref[...]
        h = jnp.maximum(dis * raw + b_ref[...], 0.0)
        o_ref[...] = (
            jnp.dot(h, w_ref[...], preferred_element_type=jnp.float32) * dis
        )

    return pl.pallas_call(
        body,
        grid=(n // _ROWBLK,),
        in_specs=[
            pl.BlockSpec((2, _ROWBLK, d), lambda i: (0, i, 0)),
            pl.BlockSpec((_ROWBLK, d), lambda i: (i, 0)),
            pl.BlockSpec((2, _ROWBLK, _LANES), lambda i: (0, i, 0)),
            pl.BlockSpec((1, d), lambda i: (0, 0)),
            pl.BlockSpec((d, dn), lambda i: (0, 0)),
        ],
        out_specs=pl.BlockSpec((_ROWBLK, dn), lambda i: (i, 0)),
        out_shape=jax.ShapeDtypeStruct((n, dn), jnp.float32),
    )(parts, y, degp, b_row, w_next)


def _tc_fin(parts, y, degp, b_row, wc, bc_row):
    n, d = y.shape
    dn = wc.shape[1]

    def body(p_ref, y_ref, dp_ref, b_ref, w_ref, bc_ref, o_ref):
        dis = _dis_block(dp_ref[...])
        p = p_ref[...]
        raw = p[0] + p[1] + y_ref[...]
        h = jnp.maximum(dis * raw + b_ref[...], 0.0)
        o_ref[...] = (
            jnp.dot(h, w_ref[...], preferred_element_type=jnp.float32)
            + bc_ref[...]
        )

    return pl.pallas_call(
        body,
        grid=(n // _ROWBLK,),
        in_specs=[
            pl.BlockSpec((2, _ROWBLK, d), lambda i: (0, i, 0)),
            pl.BlockSpec((_ROWBLK, d), lambda i: (i, 0)),
            pl.BlockSpec((2, _ROWBLK, _LANES), lambda i: (0, i, 0)),
            pl.BlockSpec((1, d), lambda i: (0, 0)),
            pl.BlockSpec((d, dn), lambda i: (0, 0)),
            pl.BlockSpec((1, dn), lambda i: (0, 0)),
        ],
        out_specs=pl.BlockSpec((_ROWBLK, dn), lambda i: (i, 0)),
        out_shape=jax.ShapeDtypeStruct((n, dn), jnp.float32),
    )(parts, y, degp, b_row, wc, bc_row)


def kernel(x, edge_index, W1, b1, W2, b2, Wc, bc):
    n, _ = x.shape
    src3d = edge_index[0].astype(jnp.int32).reshape(_NW, -1, _CHUNK)
    dst3d = edge_index[1].astype(jnp.int32).reshape(_NW, -1, _CHUNK)

    degp = _sc_degree(dst3d, n)
    y1 = _tc_y1(x, W1, degp)
    p1 = _sc_propagate(y1, src3d, dst3d)
    y2 = _tc_mid(p1, y1, degp, b1.reshape(1, -1), W2)
    p2 = _sc_propagate(y2, src3d, dst3d)
    return _tc_fin(p2, y2, degp, b2.reshape(1, -1), Wc, bc.reshape(1, -1))


# speedup vs baseline: 22.3924x; 1.1463x over previous
import functools

import jax
import jax.numpy as jnp
from jax import lax
from jax.experimental import pallas as pl
from jax.experimental.pallas import tpu as pltpu
from jax.experimental.pallas import tpu_sc as plsc

_NC = 2
_NS = 16
_NW = _NC * _NS
_LANES = 16
_CHUNK = 80
_PCHUNK = 128
_ZBLK = 80
_ROWBLK = 1000


def _sc_mesh():
    return plsc.VectorSubcoreMesh(core_axis_name="c", subcore_axis_name="s")


def _sc_degree(dst3d, n):
    rpt = dst3d.shape[1]
    nzb = n // _CHUNK
    kblocks = (nzb + _NS - 1) // _NS

    @functools.partial(
        pl.kernel,
        out_type=jax.ShapeDtypeStruct((_NC, n, _LANES), jnp.float32),
        mesh=_sc_mesh(),
        scratch_types=[
            pltpu.VMEM((rpt, _CHUNK), jnp.int32),
            pltpu.VMEM((_CHUNK, _LANES), jnp.float32),
            pltpu.VMEM((_CHUNK, _LANES), jnp.float32),
            pltpu.VMEM_SHARED((n, _LANES), jnp.float32),
        ],
    )
    def deg_kernel(dst_hbm, out_hbm, idx_v, ones_v, zero_v, acc_sh):
        cid = lax.axis_index("c")
        sid = lax.axis_index("s")
        wid = cid * _NS + sid

        @pl.loop(0, _CHUNK)
        def _(i):
            ones_v[i, :] = jnp.full((_LANES,), 1.0, jnp.float32)
            zero_v[i, :] = jnp.zeros((_LANES,), jnp.float32)

        for k in range(kblocks):
            b = sid + _NS * k

            @pl.when(b < nzb)
            def _():
                pltpu.sync_copy(zero_v, acc_sh.at[pl.ds(b * _CHUNK, _CHUNK)])

        plsc.subcore_barrier()

        pltpu.sync_copy(dst_hbm.at[wid], idx_v)

        @pl.loop(0, rpt)
        def _(j):
            pltpu.sync_copy(ones_v, acc_sh.at[idx_v.at[j]], add=True)

        plsc.subcore_barrier()

        for k in range(kblocks):
            b = sid + _NS * k

            @pl.when(b < nzb)
            def _():
                pltpu.sync_copy(
                    acc_sh.at[pl.ds(b * _CHUNK, _CHUNK)],
                    out_hbm.at[cid].at[pl.ds(b * _CHUNK, _CHUNK)],
                )

    return deg_kernel(dst3d)


def _sc_propagate(y, srcp, dstp, nch):
    n, d = y.shape
    base = nch // _NW
    extra = nch - base * _NW
    slab = base + (1 if extra else 0)
    nzb = n // _ZBLK
    kblocks = (nzb + _NS - 1) // _NS
    mmain = ((base - 1) // 8) * 8

    @functools.partial(
        pl.kernel,
        out_type=jax.ShapeDtypeStruct((_NC, n, d), jnp.float32),
        mesh=_sc_mesh(),
        scratch_types=[
            pltpu.VMEM((slab + 9, _PCHUNK), jnp.int32),
            pltpu.VMEM((slab + 9, _PCHUNK), jnp.int32),
            pltpu.VMEM((_PCHUNK, d), jnp.float32),
            pltpu.VMEM((_PCHUNK, d), jnp.float32),
            pltpu.VMEM_SHARED((n, d), jnp.float32),
        ],
    )
    def prop_kernel(
        y_hbm, src_hbm, dst_hbm, out_hbm,
        src_v, dst_v, rows_a, rows_b, acc_sh,
    ):
        cid = lax.axis_index("c")
        sid = lax.axis_index("s")
        wid = cid * _NS + sid
        start = wid * base + jnp.maximum(wid - (_NW - extra), 0)
        count = jnp.where(wid >= _NW - extra, base + 1, base)
        a8 = (start // 8) * 8
        off = start - a8

        pltpu.sync_copy(src_hbm.at[pl.ds(a8, slab + 9)], src_v)
        pltpu.sync_copy(dst_hbm.at[pl.ds(a8, slab + 9)], dst_v)

        @pl.loop(0, _PCHUNK)
        def _(i):
            for j in range(d // _LANES):
                rows_a[i, pl.ds(j * _LANES, _LANES)] = jnp.zeros(
                    (_LANES,), jnp.float32
                )

        for k in range(kblocks):
            b = sid + _NS * k

            @pl.when(b < nzb)
            def _():
                pltpu.sync_copy(
                    rows_a.at[pl.ds(0, _ZBLK)],
                    acc_sh.at[pl.ds(b * _ZBLK, _ZBLK)],
                )

        plsc.subcore_barrier()

        bufs = (rows_a, rows_b)

        @pl.loop(0, count)
        def _(k):
            pltpu.sync_copy(y_hbm.at[src_v.at[off + k]], rows_a)
            pltpu.sync_copy(rows_a, acc_sh.at[dst_v.at[off + k]], add=True)

        plsc.subcore_barrier()

        for k in range(kblocks):
            b = sid + _NS * k

            @pl.when(b < nzb)
            def _():
                pltpu.sync_copy(
                    acc_sh.at[pl.ds(b * _ZBLK, _ZBLK)],
                    out_hbm.at[cid].at[pl.ds(b * _ZBLK, _ZBLK)],
                )

    return prop_kernel(y, srcp, dstp)


def _dis_block(degp):
    deg = degp[0, :, 0:1] + degp[1, :, 0:1] + 1.0
    return lax.rsqrt(deg)


def _tc_y1(x, w1, degp):
    n, d = x.shape
    dn = w1.shape[1]

    def body(x_ref, w_ref, dp_ref, o_ref):
        dis = _dis_block(dp_ref[...])
        o_ref[...] = (
            jnp.dot(x_ref[...], w_ref[...], preferred_element_type=jnp.float32)
            * dis
        )

    return pl.pallas_call(
        body,
        grid=(n // _ROWBLK,),
        in_specs=[
            pl.BlockSpec((_ROWBLK, d), lambda i: (i, 0)),
            pl.BlockSpec((d, dn), lambda i: (0, 0)),
            pl.BlockSpec((2, _ROWBLK, _LANES), lambda i: (0, i, 0)),
        ],
        out_specs=pl.BlockSpec((_ROWBLK, dn), lambda i: (i, 0)),
        out_shape=jax.ShapeDtypeStruct((n, dn), jnp.float32),
    )(x, w1, degp)


def _tc_mid(parts, y, degp, b_row, w_next):
    n, d = y.shape
    dn = w_next.shape[1]

    def body(p_ref, y_ref, dp_ref, b_ref, w_ref, o_ref):
        dis = _dis_block(dp_ref[...])
        p = p_ref[...]
        raw = p[0] + p[1] + y_ref[...]
        hidden = jnp.maximum(dis * raw + b_ref[...], 0.0)
        o_ref[...] = (
            jnp.dot(hidden, w_ref[...], preferred_element_type=jnp.float32)
            * dis
        )

    return pl.pallas_call(
        body,
        grid=(n // _ROWBLK,),
        in_specs=[
            pl.BlockSpec((2, _ROWBLK, d), lambda i: (0, i, 0)),
            pl.BlockSpec((_ROWBLK, d), lambda i: (i, 0)),
            pl.BlockSpec((2, _ROWBLK, _LANES), lambda i: (0, i, 0)),
            pl.BlockSpec((1, d), lambda i: (0, 0)),
            pl.BlockSpec((d, dn), lambda i: (0, 0)),
        ],
        out_specs=pl.BlockSpec((_ROWBLK, dn), lambda i: (i, 0)),
        out_shape=jax.ShapeDtypeStruct((n, dn), jnp.float32),
    )(parts, y, degp, b_row, w_next)


def _tc_fin(parts, y, degp, b_row, wc, bc_row):
    n, d = y.shape
    dn = wc.shape[1]

    def body(p_ref, y_ref, dp_ref, b_ref, w_ref, bc_ref, o_ref):
        dis = _dis_block(dp_ref[...])
        p = p_ref[...]
        raw = p[0] + p[1] + y_ref[...]
        hidden = jnp.maximum(dis * raw + b_ref[...], 0.0)
        o_ref[...] = (
            jnp.dot(hidden, w_ref[...], preferred_element_type=jnp.float32)
            + bc_ref[...]
        )

    return pl.pallas_call(
        body,
        grid=(n // _ROWBLK,),
        in_specs=[
            pl.BlockSpec((2, _ROWBLK, d), lambda i: (0, i, 0)),
            pl.BlockSpec((_ROWBLK, d), lambda i: (i, 0)),
            pl.BlockSpec((2, _ROWBLK, _LANES), lambda i: (0, i, 0)),
            pl.BlockSpec((1, d), lambda i: (0, 0)),
            pl.BlockSpec((d, dn), lambda i: (0, 0)),
            pl.BlockSpec((1, dn), lambda i: (0, 0)),
        ],
        out_specs=pl.BlockSpec((_ROWBLK, dn), lambda i: (i, 0)),
        out_shape=jax.ShapeDtypeStruct((n, dn), jnp.float32),
    )(parts, y, degp, b_row, wc, bc_row)


def kernel(x, edge_index, W1, b1, W2, b2, Wc, bc):
    n, _ = x.shape
    src = edge_index[0].astype(jnp.int32)
    dst = edge_index[1].astype(jnp.int32)
    dst3d = dst.reshape(_NW, -1, _CHUNK)
    srcp = jnp.pad(src.reshape(-1, _PCHUNK), ((0, 16), (0, 0)))
    dstp = jnp.pad(dst.reshape(-1, _PCHUNK), ((0, 16), (0, 0)))

    nch = src.shape[0] // _PCHUNK

    degp = _sc_degree(dst3d, n)
    y1 = _tc_y1(x, W1, degp)
    p1 = _sc_propagate(y1, srcp, dstp, nch)
    y2 = _tc_mid(p1, y1, degp, b1.reshape(1, -1), W2)
    p2 = _sc_propagate(y2, srcp, dstp, nch)
    return _tc_fin(p2, y2, degp, b2.reshape(1, -1), Wc, bc.reshape(1, -1))
